# baseline (speedup 1.0000x reference)
import functools

import jax
import jax.numpy as jnp
from jax.experimental import pallas as pl

B, N, D = 4096, 1024, 64
BLOCK_B = 512


def _read_kernel(q_ref, mem_ref, wq_ref, bq_ref, out_ref, *, scale):
    q = q_ref[...]
    mem = mem_ref[...]
    wq = wq_ref[...]
    bq = bq_ref[...]

    Q = jax.lax.dot_general(
        q, wq, (((1,), (1,)), ((), ())), preferred_element_type=jnp.float32
    ) + bq

    s = jax.lax.dot_general(
        Q, mem, (((1,), (1,)), ((), ())), preferred_element_type=jnp.float32
    ) * scale

    m = jnp.max(s, axis=-1, keepdims=True)
    e = jnp.exp(s - m)
    denom = jnp.sum(e, axis=-1, keepdims=True)

    acc = jax.lax.dot_general(
        e, mem, (((1,), (0,)), ((), ())), preferred_element_type=jnp.float32
    )
    out_ref[...] = acc / denom


@jax.jit
def kernel(query, memory, Wq, bq):
    mem = memory[0]
    bq2 = bq.reshape(1, D)
    scale = 1.0 / (D ** 0.5)
    grid = (B // BLOCK_B,)
    return pl.pallas_call(
        functools.partial(_read_kernel, scale=scale),
        grid=grid,
        in_specs=[
            pl.BlockSpec((BLOCK_B, D), lambda i: (i, 0)),
            pl.BlockSpec((N, D), lambda i: (0, 0)),
            pl.BlockSpec((D, D), lambda i: (0, 0)),
            pl.BlockSpec((1, D), lambda i: (0, 0)),
        ],
        out_specs=pl.BlockSpec((BLOCK_B, D), lambda i: (i, 0)),
        out_shape=jax.ShapeDtypeStruct((B, D), jnp.float32),
    )(query, mem, Wq, bq2)


# speedup vs baseline: 1.0272x; 1.0272x over previous
import functools

import jax
import jax.numpy as jnp
from jax.experimental import pallas as pl

B, N, D = 4096, 1024, 64
BLOCK_B = 2048


def _read_kernel(q_ref, mem_ref, wq_ref, bq_ref, out_ref, *, scale):
    q = q_ref[...]
    mem = mem_ref[...]
    wq = wq_ref[...]
    bq = bq_ref[...]

    Q = jax.lax.dot_general(
        q, wq, (((1,), (1,)), ((), ())), preferred_element_type=jnp.float32
    ) + bq

    s = jax.lax.dot_general(
        Q, mem, (((1,), (1,)), ((), ())), preferred_element_type=jnp.float32
    ) * scale

    m = jnp.max(s, axis=-1, keepdims=True)
    e = jnp.exp(s - m)
    denom = jnp.sum(e, axis=-1, keepdims=True)

    acc = jax.lax.dot_general(
        e, mem, (((1,), (0,)), ((), ())), preferred_element_type=jnp.float32
    )
    out_ref[...] = acc / denom


@jax.jit
def kernel(query, memory, Wq, bq):
    mem = memory[0]
    bq2 = bq.reshape(1, D)
    scale = 1.0 / (D ** 0.5)
    grid = (B // BLOCK_B,)
    return pl.pallas_call(
        functools.partial(_read_kernel, scale=scale),
        grid=grid,
        in_specs=[
            pl.BlockSpec((BLOCK_B, D), lambda i: (i, 0)),
            pl.BlockSpec((N, D), lambda i: (0, 0)),
            pl.BlockSpec((D, D), lambda i: (0, 0)),
            pl.BlockSpec((1, D), lambda i: (0, 0)),
        ],
        out_specs=pl.BlockSpec((BLOCK_B, D), lambda i: (i, 0)),
        out_shape=jax.ShapeDtypeStruct((B, D), jnp.float32),
    )(query, mem, Wq, bq2)
